# baseline (speedup 1.0000x reference)
import functools

import jax
import jax.numpy as jnp
from jax import lax
from jax.experimental import pallas as pl
from jax.experimental.pallas import tpu as pltpu
from jax.experimental.pallas import tpu_sc as plsc

B = 16384
D_ITEM, D_USER, D_CAT, D_AGE = 20, 20, 17, 7
D_OUT = D_ITEM + D_USER + D_CAT + D_AGE + 1
V_ITEM, V_USER, V_CAT, V_AGE = 1000001, 1000001, 100001, 101
N_GATHER = D_ITEM + D_USER + D_CAT

_info = plsc.get_sparse_core_info()
NC, NS, L = _info.num_cores, _info.num_subcores, _info.num_lanes
NW = NC * NS
BPW = B // NW
NG = BPW // L

_mesh = plsc.VectorSubcoreMesh(core_axis_name="c", subcore_axis_name="s")

_scratch = (
    [pltpu.VMEM((BPW,), jnp.int32)] * 4
    + [pltpu.VMEM((V_AGE * D_AGE,), jnp.float32)]
    + [pltpu.VMEM((BPW,), jnp.float32)]
    + [pltpu.VMEM((BPW,), jnp.int32)] * N_GATHER
    + [pltpu.VMEM((BPW,), jnp.float32)] * N_GATHER
    + [pltpu.VMEM((D_OUT, BPW), jnp.float32)]
    + [pltpu.SemaphoreType.DMA]
)


@functools.partial(
    pl.kernel,
    mesh=_mesh,
    compiler_params=pltpu.CompilerParams(use_tc_tiling_on_sc=True,
                                         needs_layout_passes=False),
    out_type=jax.ShapeDtypeStruct((D_OUT, B), jnp.float32),
    scratch_types=_scratch,
)
def _emb_kernel(item_hbm, user_hbm, cat_hbm, age_hbm, price_hbm,
                F_item_hbm, F_user_hbm, F_cat_hbm, F_age_hbm, out_hbm,
                *scratch):
    idx_item, idx_user, idx_cat, idx_age, age_tab, price_v = scratch[:6]
    e_refs = scratch[6:6 + N_GATHER]
    c_refs = scratch[6 + N_GATHER:6 + 2 * N_GATHER]
    blockT = scratch[6 + 2 * N_GATHER]
    sem = scratch[7 + 2 * N_GATHER]

    wid = lax.axis_index("s") * NC + lax.axis_index("c")
    base = wid * BPW

    pltpu.sync_copy(item_hbm.at[pl.ds(base, BPW)], idx_item)
    pltpu.sync_copy(user_hbm.at[pl.ds(base, BPW)], idx_user)
    pltpu.sync_copy(cat_hbm.at[pl.ds(base, BPW)], idx_cat)
    pltpu.sync_copy(age_hbm.at[pl.ds(base, BPW)], idx_age)
    pltpu.sync_copy(F_age_hbm, age_tab)

    feats = ((idx_item, D_ITEM, V_ITEM), (idx_user, D_USER, V_USER),
             (idx_cat, D_CAT, V_CAT))

    def build_body(g, _):
        sl = pl.ds(g * L, L)
        col = 0
        for idx_ref, width, vocab in feats:
            v = idx_ref[sl]
            for j in range(width):
                e_refs[col][sl] = v + (j * vocab)
                col += 1
        return _

    lax.fori_loop(0, NG, build_body, 0)

    copies = []
    col = 0
    for F_hbm, (_, width, _v) in ((F_item_hbm, feats[0]),
                                  (F_user_hbm, feats[1]),
                                  (F_cat_hbm, feats[2])):
        for _ in range(width):
            copies.append(pltpu.make_async_copy(
                F_hbm.at[e_refs[col]], c_refs[col], sem))
            col += 1
    for c in copies:
        c.start()

    pltpu.sync_copy(price_hbm.at[pl.ds(base, BPW)], price_v)

    def age_body(g, _):
        sl = pl.ds(g * L, L)
        a0 = idx_age[sl]
        for j in range(D_AGE):
            v = plsc.load_gather(age_tab, [a0 + j * V_AGE])
            blockT[N_GATHER + j, sl] = v
        blockT[D_OUT - 1, sl] = price_v[sl]
        return _

    lax.fori_loop(0, NG, age_body, 0)

    for c in copies:
        c.wait()

    def asm_body(g, _):
        sl = pl.ds(g * L, L)
        for c in range(N_GATHER):
            blockT[c, sl] = c_refs[c][sl]
        return _

    lax.fori_loop(0, NG, asm_body, 0)

    pltpu.sync_copy(blockT, out_hbm.at[:, pl.ds(base, BPW)])


def kernel(cat_item_id, cat_user_id, cat_category, disc_clip_age,
           norm_clip_price, W_item, W_user, W_cat, W_age):
    out_t = _emb_kernel(
        cat_item_id.astype(jnp.int32),
        cat_user_id.astype(jnp.int32),
        cat_category.astype(jnp.int32),
        disc_clip_age.astype(jnp.int32),
        norm_clip_price,
        W_item.T.reshape(-1),
        W_user.T.reshape(-1),
        W_cat.T.reshape(-1),
        W_age.T.reshape(-1),
    )
    return out_t.T


# speedup vs baseline: 2.2950x; 2.2950x over previous
import functools

import jax
import jax.numpy as jnp
from jax import lax
from jax.experimental import pallas as pl
from jax.experimental.pallas import tpu as pltpu
from jax.experimental.pallas import tpu_sc as plsc

B = 16384
D_ITEM, D_USER, D_CAT, D_AGE = 20, 20, 17, 7
D_OUT = D_ITEM + D_USER + D_CAT + D_AGE + 1
V_ITEM, V_USER, V_CAT, V_AGE = 1000001, 1000001, 100001, 101
N_GATHER = D_ITEM + D_USER + D_CAT

_info = plsc.get_sparse_core_info()
NC, NS, L = _info.num_cores, _info.num_subcores, _info.num_lanes
NW = NC * NS
BPW = B // NW
NG = BPW // L

_mesh = plsc.VectorSubcoreMesh(core_axis_name="c", subcore_axis_name="s")

_scratch = (
    [pltpu.VMEM((BPW,), jnp.int32)] * 4
    + [pltpu.VMEM((V_AGE * D_AGE,), jnp.float32)]
    + [pltpu.VMEM((BPW,), jnp.float32)]
    + [pltpu.VMEM((BPW,), jnp.int32)] * N_GATHER
    + [pltpu.VMEM((BPW,), jnp.float32)] * N_GATHER
    + [pltpu.VMEM((D_OUT, BPW), jnp.float32)]
    + [pltpu.SemaphoreType.DMA]
)


@functools.partial(
    pl.kernel,
    mesh=_mesh,
    compiler_params=pltpu.CompilerParams(use_tc_tiling_on_sc=True,
                                         needs_layout_passes=False),
    out_type=jax.ShapeDtypeStruct((D_OUT, B), jnp.float32),
    scratch_types=_scratch,
)
def _emb_kernel(item_hbm, user_hbm, cat_hbm, age_hbm, price_hbm,
                F_item_hbm, F_user_hbm, F_cat_hbm, F_age_hbm, out_hbm,
                *scratch):
    idx_item, idx_user, idx_cat, idx_age, age_tab, price_v = scratch[:6]
    e_refs = scratch[6:6 + N_GATHER]
    c_refs = scratch[6 + N_GATHER:6 + 2 * N_GATHER]
    blockT = scratch[6 + 2 * N_GATHER]
    sem = scratch[7 + 2 * N_GATHER]

    wid = lax.axis_index("s") * NC + lax.axis_index("c")
    base = wid * BPW

    pltpu.sync_copy(item_hbm.at[pl.ds(base, BPW)], idx_item)
    pltpu.sync_copy(user_hbm.at[pl.ds(base, BPW)], idx_user)
    pltpu.sync_copy(cat_hbm.at[pl.ds(base, BPW)], idx_cat)
    pltpu.sync_copy(age_hbm.at[pl.ds(base, BPW)], idx_age)
    pltpu.sync_copy(F_age_hbm, age_tab)

    feats = ((idx_item, D_ITEM), (idx_user, D_USER), (idx_cat, D_CAT))

    def build_body(g, _):
        sl = pl.ds(g * L, L)
        col = 0
        for idx_ref, width in feats:
            v = idx_ref[sl] * width
            for j in range(width):
                e_refs[col][sl] = v + j
                col += 1
        return _

    lax.fori_loop(0, NG, build_body, 0)

    copies = []
    col = 0
    for F_hbm, (_, width) in ((F_item_hbm, feats[0]),
                              (F_user_hbm, feats[1]),
                              (F_cat_hbm, feats[2])):
        for _ in range(width):
            copies.append(pltpu.make_async_copy(
                F_hbm.at[e_refs[col]], c_refs[col], sem))
            col += 1
    for c in copies:
        c.start()

    pltpu.sync_copy(price_hbm.at[pl.ds(base, BPW)], price_v)

    def age_body(g, _):
        sl = pl.ds(g * L, L)
        a0 = idx_age[sl] * D_AGE
        for j in range(D_AGE):
            v = plsc.load_gather(age_tab, [a0 + j])
            blockT[N_GATHER + j, sl] = v
        blockT[D_OUT - 1, sl] = price_v[sl]
        return _

    lax.fori_loop(0, NG, age_body, 0)

    for c in copies:
        c.wait()

    def asm_body(g, _):
        sl = pl.ds(g * L, L)
        for c in range(N_GATHER):
            blockT[c, sl] = c_refs[c][sl]
        return _

    lax.fori_loop(0, NG, asm_body, 0)

    pltpu.sync_copy(blockT, out_hbm.at[:, pl.ds(base, BPW)])


def kernel(cat_item_id, cat_user_id, cat_category, disc_clip_age,
           norm_clip_price, W_item, W_user, W_cat, W_age):
    out_t = _emb_kernel(
        cat_item_id.astype(jnp.int32),
        cat_user_id.astype(jnp.int32),
        cat_category.astype(jnp.int32),
        disc_clip_age.astype(jnp.int32),
        norm_clip_price,
        W_item.reshape(-1),
        W_user.reshape(-1),
        W_cat.reshape(-1),
        W_age.reshape(-1),
    )
    return out_t.T
